# baseline (speedup 1.0000x reference)
import functools

import jax
import jax.numpy as jnp
from jax import lax
from jax.experimental import pallas as pl
from jax.experimental.pallas import tpu as pltpu
from jax.experimental.pallas import tpu_sc as plsc

B, T, V, H = 1024, 50, 1000, 1024
L = 16
TP = 64


def _make_sc_kernel():
    info = plsc.get_sparse_core_info()
    nc, ns = info.num_cores, info.num_subcores
    nw = nc * ns
    bpw = B // nw

    mesh = plsc.VectorSubcoreMesh(core_axis_name="c", subcore_axis_name="s")

    @functools.partial(
        pl.kernel,
        mesh=mesh,
        out_type=jax.ShapeDtypeStruct((B, H), jnp.float32),
        scratch_types=[
            pltpu.VMEM((bpw, TP), jnp.int32),
            pltpu.VMEM((TP, H), jnp.float32),
            pltpu.VMEM((H,), jnp.float32),
            pltpu.SemaphoreType.DMA,
        ],
    )
    def k(idx_hbm, cb_hbm, out_hbm, idx_v, rows_v, acc_v, sem):
        wid = lax.axis_index("s") * nc + lax.axis_index("c")
        base = wid * bpw
        pltpu.sync_copy(idx_hbm.at[pl.ds(base, bpw)], idx_v)

        def row(r, _):
            pltpu.async_copy(cb_hbm.at[idx_v.at[r]], rows_v, sem).wait()

            def chunk(kk, _):
                def tok(j, acc):
                    return acc + rows_v[j, pl.ds(kk * L, L)]

                accv = lax.fori_loop(0, T, tok, jnp.zeros((L,), jnp.float32))
                acc_v[pl.ds(kk * L, L)] = accv
                return 0

            lax.fori_loop(0, H // L, chunk, 0)
            pltpu.sync_copy(acc_v, out_hbm.at[base + r])
            return 0

        lax.fori_loop(0, bpw, row, 0)

    return k


_sc_kernel = _make_sc_kernel()


def kernel(indices, codebook):
    idx = jnp.pad(indices.astype(jnp.int32), ((0, 0), (0, TP - T)))
    return _sc_kernel(idx, codebook)


# speedup vs baseline: 18.5710x; 18.5710x over previous
import functools

import jax
import jax.numpy as jnp
from jax import lax
from jax.experimental import pallas as pl
from jax.experimental.pallas import tpu as pltpu
from jax.experimental.pallas import tpu_sc as plsc

B, T, V, H = 1024, 50, 1000, 1024
L = 16


def _make_counts_kernel():
    info = plsc.get_sparse_core_info()
    nc, ns = info.num_cores, info.num_subcores
    nw = nc * ns
    bpw = B // nw
    ng = bpw // L

    mesh = plsc.VectorSubcoreMesh(core_axis_name="c", subcore_axis_name="s")

    @functools.partial(
        pl.kernel,
        mesh=mesh,
        compiler_params=pltpu.CompilerParams(
            needs_layout_passes=False,
            use_tc_tiling_on_sc=False,
        ),
        out_type=jax.ShapeDtypeStruct((nw, bpw * V), jnp.float32),
        scratch_types=[
            pltpu.VMEM((ng, T, L), jnp.int32),
            pltpu.VMEM((bpw * V,), jnp.float32),
        ],
    )
    def k(idx_hbm, cnt_hbm, idx_v, cnt_v):
        wid = lax.axis_index("s") * nc + lax.axis_index("c")
        pltpu.sync_copy(idx_hbm.at[pl.ds(wid * ng, ng)], idx_v)

        zeros = jnp.zeros((L,), jnp.float32)

        def zloop(i, _):
            for u in range(8):
                cnt_v[pl.ds((i * 8 + u) * L, L)] = zeros
            return 0

        lax.fori_loop(0, bpw * V // (8 * L), zloop, 0)

        ones = jnp.ones((L,), jnp.float32)
        lane_base = lax.iota(jnp.int32, L) * V
        for g in range(ng):
            gbase = g * L * V
            for j in range(T):
                addr = idx_v[g, j, :] + lane_base + gbase
                plsc.addupdate_scatter(cnt_v, [addr], ones)

        pltpu.sync_copy(cnt_v, cnt_hbm.at[wid])

    return k


_counts_kernel = _make_counts_kernel()


def _mm_body(a_ref, b_ref, o_ref):
    o_ref[...] = jnp.dot(a_ref[...], b_ref[...],
                         preferred_element_type=jnp.float32)


_BM = 256
_matmul = pl.pallas_call(
    _mm_body,
    grid=(B // _BM,),
    in_specs=[
        pl.BlockSpec((_BM, V), lambda i: (i, 0)),
        pl.BlockSpec((V, H), lambda i: (0, 0)),
    ],
    out_specs=pl.BlockSpec((_BM, H), lambda i: (i, 0)),
    out_shape=jax.ShapeDtypeStruct((B, H), jnp.float32),
)


def kernel(indices, codebook):
    idx_lane = indices.astype(jnp.int32).reshape(B // L, L, T).transpose(0, 2, 1)
    counts = _counts_kernel(idx_lane).reshape(B, V)
    return _matmul(counts, codebook)
